# Initial kernel scaffold; baseline (speedup 1.0000x reference)
import jax
import jax.numpy as jnp
from jax.experimental import pallas as pl


def kernel(x, domain_emb_weight):
    raise NotImplementedError("write your pallas kernel here")


# speedup vs baseline: 1.4998x; 1.4998x over previous
import functools

import jax
import jax.numpy as jnp
from jax import lax
from jax.experimental import pallas as pl
from jax.experimental.pallas import tpu as pltpu
from jax.experimental.pallas import tpu_sc as plsc

BATCH = 16384
MAX_D = 20
DIM = 32

_B = BATCH * MAX_D
_NW = 32
_PER_W = _B // _NW
_BLK = 128
_NBLK = _PER_W // _BLK
_GRP = 8
_NGRP = _NBLK // _GRP


def _emb_body(idx_hbm, table_hbm, out_hbm, idx_v, rows_v, gsem, osem):
    cid = lax.axis_index("c")
    sid = lax.axis_index("s")
    wid = sid * 2 + cid

    pltpu.sync_copy(idx_hbm.at[wid], idx_v)

    def group(g, carry):
        copies = []
        for j in range(_GRP):
            copies.append(
                pltpu.async_copy(
                    table_hbm.at[idx_v.at[g * _GRP + j]], rows_v.at[j], gsem
                )
            )
        for c in copies:
            c.wait()
        pltpu.async_copy(rows_v, out_hbm.at[wid, g], osem).wait()
        return carry

    lax.fori_loop(0, _NGRP, group, 0)


@jax.jit
def _emb_call(x_flat, table):
    mesh = plsc.VectorSubcoreMesh(core_axis_name="c", subcore_axis_name="s")
    f = pl.kernel(
        _emb_body,
        out_type=jax.ShapeDtypeStruct((_NW, _NGRP, _GRP, _BLK, DIM), jnp.float32),
        mesh=mesh,
        scratch_types=[
            pltpu.VMEM((_NBLK, _BLK), jnp.int32),
            pltpu.VMEM((_GRP, _BLK, DIM), jnp.float32),
            pltpu.SemaphoreType.DMA,
            pltpu.SemaphoreType.DMA,
        ],
        compiler_params=pltpu.CompilerParams(use_tc_tiling_on_sc=False),
    )
    return f(x_flat, table)


def kernel(x, domain_emb_weight):
    x_flat = x.reshape(_NW, _NBLK, _BLK).astype(jnp.int32)
    out = _emb_call(x_flat, domain_emb_weight)
    return out.reshape(BATCH, MAX_D, DIM)
